# baseline (speedup 1.0000x reference)
import jax
import jax.numpy as jnp
from jax import lax
from jax.experimental import pallas as pl
from jax.experimental.pallas import tpu as pltpu
from jax.experimental.pallas import tpu_sc as plsc

SEQ_LEN = 200
N_DIMS = 64
BATCH = 4096
PAIR = 2 * N_DIMS
T_PAIRS = SEQ_LEN // 2

NUM_CORES = 2
NUM_SUBCORES = 16
NUM_WORKERS = NUM_CORES * NUM_SUBCORES
BPW = BATCH // NUM_WORKERS

NBUF_G = 4
NBUF_O = 2
SCR = 256


def _body(resp_hbm, pos_hbm, tab_hbm, out_hbm, idx_ring, gbuf, obuf, pos_v,
          sem_i, sem_g, sem_o):
    wid = lax.axis_index("s") * NUM_CORES + lax.axis_index("c")
    b0 = wid * BPW

    pltpu.sync_copy(pos_hbm, pos_v)

    def idx_desc(s, kg):
        return pltpu.make_async_copy(
            resp_hbm.at[pl.ds(s * BATCH + b0, BPW)],
            idx_ring.at[pl.ds(kg * SCR, BPW)],
            sem_i.at[kg],
        )

    def gather_desc(kg):
        return pltpu.make_async_copy(
            tab_hbm.at[idx_ring.at[pl.ds(kg * SCR, BPW)]],
            gbuf.at[kg],
            sem_g.at[kg],
        )

    def out_slice(t):
        return out_hbm.at[pl.ds(b0, BPW), pl.ds(t * PAIR, PAIR)]

    for s in range(2):
        idx_desc(s, s).start()
        idx_desc(s, s).wait()
    gather_desc(0).start()
    gather_desc(1).start()
    idx_desc(2, 2).start()
    idx_desc(3, 3).start()

    def do_pair(t, ka, kb, ko, last):
        gather_desc(ka).wait()
        gather_desc(kb).wait()

        @pl.when(t - NBUF_O >= 0)
        def _():
            pltpu.make_async_copy(
                obuf.at[ko], out_slice(t - NBUF_O), sem_o.at[ko]
            ).wait()

        if not last:
            ka_n, kb_n = (ka + 2) % NBUF_G, (kb + 2) % NBUF_G

            @pl.when(t + 1 <= T_PAIRS - 1)
            def _():
                idx_desc(2 * t + 2, ka_n).wait()
                idx_desc(2 * t + 3, kb_n).wait()
                gather_desc(ka_n).start()
                gather_desc(kb_n).start()

            @pl.when(t + 2 <= T_PAIRS - 1)
            def _():
                idx_desc(2 * t + 4, ka).start()
                idx_desc(2 * t + 5, kb).start()

        pv = [
            pos_v[pl.ds(2 * t * N_DIMS + 16 * j, 16)]
            for j in range(PAIR // 16)
        ]

        def add_row(b, c_):
            for h, kg in ((0, ka), (1, kb)):
                for l in range(N_DIMS // 16):
                    obuf[ko, b, pl.ds(h * N_DIMS + 16 * l, 16)] = (
                        gbuf[kg, b, pl.ds(16 * l, 16)]
                        + pv[h * (N_DIMS // 16) + l]
                    )
            return c_

        lax.fori_loop(0, BPW, add_row, 0, unroll=4)
        pltpu.async_copy(obuf.at[ko], out_slice(t), sem_o.at[ko])

    def step(i, carry):
        for k2 in range(2):
            t = i * 2 + k2
            ka = (2 * k2) % NBUF_G
            kb = (2 * k2 + 1) % NBUF_G
            ko = k2 % NBUF_O
            do_pair(t, ka, kb, ko, last=False)
        return carry

    lax.fori_loop(0, T_PAIRS // 2, step, 0)
    for t in range(T_PAIRS - NBUF_O, T_PAIRS):
        ko = t % NBUF_O
        pltpu.make_async_copy(obuf.at[ko], out_slice(t), sem_o.at[ko]).wait()


@jax.jit
def _run(resp_flat, pos_flat, tab_pad):
    mesh = plsc.VectorSubcoreMesh(core_axis_name="c", subcore_axis_name="s")
    f = pl.kernel(
        _body,
        out_type=jax.ShapeDtypeStruct((BATCH, SEQ_LEN * N_DIMS), jnp.float32),
        mesh=mesh,
        scratch_types=[
            pltpu.VMEM((NBUF_G * SCR,), jnp.int32),
            pltpu.VMEM((NBUF_G, BPW, PAIR), jnp.float32),
            pltpu.VMEM((NBUF_O, BPW, PAIR), jnp.float32),
            pltpu.VMEM((SEQ_LEN * N_DIMS,), jnp.float32),
            pltpu.SemaphoreType.DMA((NBUF_G,)),
            pltpu.SemaphoreType.DMA((NBUF_G,)),
            pltpu.SemaphoreType.DMA((NBUF_O,)),
        ],
        compiler_params=pltpu.CompilerParams(use_tc_tiling_on_sc=True),
    )
    out = f(resp_flat, pos_flat, tab_pad)
    return out.reshape(BATCH, SEQ_LEN, N_DIMS)


def kernel(response, position_embed, response_embed):
    resp_flat = response.astype(jnp.int32).T.reshape(-1)
    pos_flat = position_embed.reshape(-1)
    tab_pad = jnp.pad(response_embed, ((0, 0), (0, PAIR - N_DIMS)))
    return _run(resp_flat, pos_flat, tab_pad)


# speedup vs baseline: 1.0044x; 1.0044x over previous
import jax
import jax.numpy as jnp
from jax import lax
from jax.experimental import pallas as pl
from jax.experimental.pallas import tpu as pltpu
from jax.experimental.pallas import tpu_sc as plsc

SEQ_LEN = 200
N_DIMS = 64
BATCH = 4096
PAIR = 2 * N_DIMS
T_PAIRS = SEQ_LEN // 2

NUM_CORES = 2
NUM_SUBCORES = 16
NUM_WORKERS = NUM_CORES * NUM_SUBCORES
BPW = BATCH // NUM_WORKERS

NBUF_G = 4
NBUF_O = 2
SCR = 256


def _body(resp_hbm, pos_hbm, tab_hbm, out_hbm, idx_ring, gbuf, obuf, pos_v,
          sem_i, sem_g, sem_o):
    wid = lax.axis_index("s") * NUM_CORES + lax.axis_index("c")
    b0 = wid * BPW

    pltpu.sync_copy(pos_hbm, pos_v)

    def idx_desc(s, kg):
        return pltpu.make_async_copy(
            resp_hbm.at[pl.ds(s * BATCH + b0, BPW)],
            idx_ring.at[pl.ds(kg * SCR, BPW)],
            sem_i.at[kg],
        )

    def gather_desc(kg):
        return pltpu.make_async_copy(
            tab_hbm.at[idx_ring.at[pl.ds(kg * SCR, BPW)]],
            gbuf.at[kg],
            sem_g.at[kg],
        )

    def out_slice(t):
        return out_hbm.at[pl.ds(b0, BPW), pl.ds(t * PAIR, PAIR)]

    for s in range(2):
        idx_desc(s, s).start()
        idx_desc(s, s).wait()
    gather_desc(0).start()
    gather_desc(1).start()
    idx_desc(2, 2).start()
    idx_desc(3, 3).start()

    def do_pair(t, ka, kb, ko, last):
        gather_desc(ka).wait()
        gather_desc(kb).wait()

        @pl.when(t - NBUF_O >= 0)
        def _():
            pltpu.make_async_copy(
                obuf.at[ko], out_slice(t - NBUF_O), sem_o.at[ko]
            ).wait()

        if not last:
            ka_n, kb_n = (ka + 2) % NBUF_G, (kb + 2) % NBUF_G

            @pl.when(t + 1 <= T_PAIRS - 1)
            def _():
                idx_desc(2 * t + 2, ka_n).wait()
                idx_desc(2 * t + 3, kb_n).wait()
                gather_desc(ka_n).start()
                gather_desc(kb_n).start()

            @pl.when(t + 2 <= T_PAIRS - 1)
            def _():
                idx_desc(2 * t + 4, ka).start()
                idx_desc(2 * t + 5, kb).start()

        pv = [
            pos_v[pl.ds(2 * t * N_DIMS + 16 * j, 16)]
            for j in range(PAIR // 16)
        ]

        def add_row(b, c_):
            for h, kg in ((0, ka), (1, kb)):
                for l in range(N_DIMS // 16):
                    obuf[ko, b, pl.ds(h * N_DIMS + 16 * l, 16)] = (
                        gbuf[kg, b, pl.ds(16 * l, 16)]
                        + pv[h * (N_DIMS // 16) + l]
                    )
            return c_

        lax.fori_loop(0, BPW, add_row, 0, unroll=2)
        pltpu.async_copy(obuf.at[ko], out_slice(t), sem_o.at[ko])

    def step(i, carry):
        for k2 in range(2):
            t = i * 2 + k2
            ka = (2 * k2) % NBUF_G
            kb = (2 * k2 + 1) % NBUF_G
            ko = k2 % NBUF_O
            do_pair(t, ka, kb, ko, last=False)
        return carry

    lax.fori_loop(0, T_PAIRS // 2, step, 0)
    for t in range(T_PAIRS - NBUF_O, T_PAIRS):
        ko = t % NBUF_O
        pltpu.make_async_copy(obuf.at[ko], out_slice(t), sem_o.at[ko]).wait()


@jax.jit
def _run(resp_flat, pos_flat, tab_pad):
    mesh = plsc.VectorSubcoreMesh(core_axis_name="c", subcore_axis_name="s")
    f = pl.kernel(
        _body,
        out_type=jax.ShapeDtypeStruct((BATCH, SEQ_LEN * N_DIMS), jnp.float32),
        mesh=mesh,
        scratch_types=[
            pltpu.VMEM((NBUF_G * SCR,), jnp.int32),
            pltpu.VMEM((NBUF_G, BPW, PAIR), jnp.float32),
            pltpu.VMEM((NBUF_O, BPW, PAIR), jnp.float32),
            pltpu.VMEM((SEQ_LEN * N_DIMS,), jnp.float32),
            pltpu.SemaphoreType.DMA((NBUF_G,)),
            pltpu.SemaphoreType.DMA((NBUF_G,)),
            pltpu.SemaphoreType.DMA((NBUF_O,)),
        ],
        compiler_params=pltpu.CompilerParams(use_tc_tiling_on_sc=True),
    )
    out = f(resp_flat, pos_flat, tab_pad)
    return out.reshape(BATCH, SEQ_LEN, N_DIMS)


def kernel(response, position_embed, response_embed):
    resp_flat = response.astype(jnp.int32).T.reshape(-1)
    pos_flat = position_embed.reshape(-1)
    tab_pad = jnp.pad(response_embed, ((0, 0), (0, PAIR - N_DIMS)))
    return _run(resp_flat, pos_flat, tab_pad)
